# baseline (speedup 1.0000x reference)
import functools

import jax
import jax.numpy as jnp
from jax import lax
from jax.experimental import pallas as pl
from jax.experimental.pallas import tpu as pltpu
from jax.experimental.pallas import tpu_sc as plsc

VOCAB = 100000
DIM = 128
B = 4096 * 50
NC = 2
NS = 16
NW = NC * NS
B_PER_W = B // NW
CHUNK = 128
NCHUNK = B_PER_W // CHUNK
NBUF = 5
PF = 3

assert NCHUNK % NBUF == 0 and PF <= NBUF - 2


def _emb_body(idx_hbm, table_hbm, out_hbm, idx_v, *rest):
  bufs = list(rest[:NBUF])
  gsem = list(rest[NBUF:2 * NBUF])
  wsem = list(rest[2 * NBUF:])
  wid = lax.axis_index("s") * NC + lax.axis_index("c")
  base = wid * B_PER_W
  pltpu.sync_copy(idx_hbm.at[pl.ds(base, B_PER_W)], idx_v)

  def start_gather(j, b):
    pltpu.async_copy(
        table_hbm.at[idx_v.at[pl.ds(j * CHUNK, CHUNK)]], bufs[b], gsem[b])

  def wait_gather(j, b):
    pltpu.make_async_copy(
        table_hbm.at[idx_v.at[pl.ds(j * CHUNK, CHUNK)]], bufs[b],
        gsem[b]).wait()

  def start_write(j, b):
    pltpu.async_copy(
        bufs[b], out_hbm.at[pl.ds(base + j * CHUNK, CHUNK)], wsem[b])

  def wait_write(j, b):
    pltpu.make_async_copy(
        bufs[b], out_hbm.at[pl.ds(base + j * CHUNK, CHUNK)], wsem[b]).wait()

  def visit(j, b, pf_wait, pf_gather):
    if pf_gather:
      bf = (b + PF) % NBUF
      if pf_wait:
        wait_write(j + PF - NBUF, bf)
      start_gather(j + PF, bf)
    wait_gather(j, b)
    start_write(j, b)

  for jf in range(PF):
    start_gather(jf, jf % NBUF)
  for b in range(NBUF):
    visit(b, b, pf_wait=(b + PF >= NBUF), pf_gather=True)

  def group(i, _):
    g = i * NBUF
    for b in range(NBUF):
      visit(g + b, b, pf_wait=True, pf_gather=True)
    return 0

  lax.fori_loop(1, NCHUNK // NBUF - 1, group, 0)

  for b in range(NBUF):
    j = NCHUNK - NBUF + b
    visit(j, b, pf_wait=True, pf_gather=(j + PF < NCHUNK))
  for b in range(NBUF):
    wait_write(NCHUNK - NBUF + b, b)


@jax.jit
def _embed(idx1d, table):
  mesh = plsc.VectorSubcoreMesh(core_axis_name="c", subcore_axis_name="s")
  k = functools.partial(
      pl.kernel,
      mesh=mesh,
      out_type=jax.ShapeDtypeStruct((B, DIM), jnp.float32),
      scratch_types=(
          [pltpu.VMEM((B_PER_W,), jnp.int32)]
          + [pltpu.VMEM((CHUNK, DIM), jnp.float32)] * NBUF
          + [pltpu.SemaphoreType.DMA] * (2 * NBUF)
      ),
  )(_emb_body)
  return k(idx1d, table)


def kernel(x, embed_weight):
  idx = x.astype(jnp.int32).reshape(B)
  out = _embed(idx, embed_weight)
  return out.reshape(x.shape[0], x.shape[1], DIM)


# speedup vs baseline: 1.7988x; 1.7988x over previous
import functools

import jax
import jax.numpy as jnp
from jax import lax
from jax.experimental import pallas as pl
from jax.experimental.pallas import tpu as pltpu
from jax.experimental.pallas import tpu_sc as plsc

VOCAB = 100000
DIM = 128
BATCH = 4096
SEQ = 50
NC = 2
NS = 16
NW = NC * NS
ROWS_PER_W = BATCH // NW
NRB = 4
NCHUNK = ROWS_PER_W // NRB
NBUF = 4
PF = 2

assert NCHUNK % NBUF == 0 and PF <= NBUF - 2


def _emb_body(idx_hbm, table_hbm, out_hbm, idx_v, *rest):
  bufs = list(rest[:NBUF])
  gsem = list(rest[NBUF:2 * NBUF])
  wsem = list(rest[2 * NBUF:])
  wid = lax.axis_index("s") * NC + lax.axis_index("c")
  base = wid * ROWS_PER_W
  pltpu.sync_copy(idx_hbm.at[pl.ds(base, ROWS_PER_W)], idx_v)

  def start_gather(j, b):
    for i in range(NRB):
      pltpu.async_copy(
          table_hbm.at[idx_v.at[j * NRB + i]], bufs[b].at[i], gsem[b])

  def wait_gather(j, b):
    for i in range(NRB):
      pltpu.make_async_copy(
          table_hbm.at[idx_v.at[j * NRB + i]], bufs[b].at[i], gsem[b]).wait()

  def start_write(j, b):
    pltpu.async_copy(
        bufs[b], out_hbm.at[pl.ds(base + j * NRB, NRB)], wsem[b])

  def wait_write(j, b):
    pltpu.make_async_copy(
        bufs[b], out_hbm.at[pl.ds(base + j * NRB, NRB)], wsem[b]).wait()

  def visit(j, b, pf_wait, pf_gather):
    if pf_gather:
      bf = (b + PF) % NBUF
      if pf_wait:
        wait_write(j + PF - NBUF, bf)
      start_gather(j + PF, bf)
    wait_gather(j, b)
    start_write(j, b)

  for jf in range(PF):
    start_gather(jf, jf % NBUF)
  for b in range(NBUF):
    visit(b, b, pf_wait=(b + PF >= NBUF), pf_gather=True)

  def group(i, _):
    g = i * NBUF
    for b in range(NBUF):
      visit(g + b, b, pf_wait=True, pf_gather=True)
    return 0

  lax.fori_loop(1, NCHUNK // NBUF - 1, group, 0)

  for b in range(NBUF):
    j = NCHUNK - NBUF + b
    visit(j, b, pf_wait=True, pf_gather=(j + PF < NCHUNK))
  for b in range(NBUF):
    wait_write(NCHUNK - NBUF + b, b)


@jax.jit
def _embed(idx, table):
  mesh = plsc.VectorSubcoreMesh(core_axis_name="c", subcore_axis_name="s")
  k = functools.partial(
      pl.kernel,
      mesh=mesh,
      out_type=jax.ShapeDtypeStruct((BATCH, SEQ, DIM), jnp.float32),
      scratch_types=(
          [pltpu.VMEM((ROWS_PER_W, SEQ), jnp.int32)]
          + [pltpu.VMEM((NRB, SEQ, DIM), jnp.float32)] * NBUF
          + [pltpu.SemaphoreType.DMA] * (2 * NBUF)
      ),
  )(_emb_body)
  return k(idx, table)


def kernel(x, embed_weight):
  return _embed(x.astype(jnp.int32), embed_weight)


# speedup vs baseline: 3.2178x; 1.7889x over previous
import functools

import jax
import jax.numpy as jnp
from jax import lax
from jax.experimental import pallas as pl
from jax.experimental.pallas import tpu as pltpu
from jax.experimental.pallas import tpu_sc as plsc

VOCAB = 100000
DIM = 128
BATCH = 4096
SEQ = 50
NC = 2
NS = 16
NW = NC * NS
COLS_PER_W = BATCH // NW
NCHUNK = SEQ
NBUF = 5
PF = 3

assert NCHUNK % NBUF == 0 and PF <= NBUF - 2


def _emb_body(idx_hbm, table_hbm, out_hbm, idx_v, *rest):
  bufs = list(rest[:NBUF])
  gsem = list(rest[NBUF:2 * NBUF])
  wsem = list(rest[2 * NBUF:])
  wid = lax.axis_index("s") * NC + lax.axis_index("c")
  col0 = wid * COLS_PER_W
  pltpu.sync_copy(idx_hbm.at[:, pl.ds(col0, COLS_PER_W)], idx_v)

  def start_gather(j, b):
    pltpu.async_copy(table_hbm.at[idx_v.at[j]], bufs[b].at[0], gsem[b])

  def wait_gather(j, b):
    pltpu.make_async_copy(
        table_hbm.at[idx_v.at[j]], bufs[b].at[0], gsem[b]).wait()

  def start_write(j, b):
    pltpu.async_copy(
        bufs[b], out_hbm.at[pl.ds(j, 1), pl.ds(col0, COLS_PER_W)], wsem[b])

  def wait_write(j, b):
    pltpu.make_async_copy(
        bufs[b], out_hbm.at[pl.ds(j, 1), pl.ds(col0, COLS_PER_W)],
        wsem[b]).wait()

  def visit(j, b, pf_wait, pf_gather):
    if pf_gather:
      bf = (b + PF) % NBUF
      if pf_wait:
        wait_write(j + PF - NBUF, bf)
      start_gather(j + PF, bf)
    wait_gather(j, b)
    start_write(j, b)

  for jf in range(PF):
    start_gather(jf, jf % NBUF)
  for b in range(NBUF):
    visit(b, b, pf_wait=(b + PF >= NBUF), pf_gather=True)

  def group(i, _):
    g = i * NBUF
    for b in range(NBUF):
      visit(g + b, b, pf_wait=True, pf_gather=True)
    return 0

  lax.fori_loop(1, NCHUNK // NBUF - 1, group, 0)

  for b in range(NBUF):
    j = NCHUNK - NBUF + b
    visit(j, b, pf_wait=True, pf_gather=(j + PF < NCHUNK))
  for b in range(NBUF):
    wait_write(NCHUNK - NBUF + b, b)


@jax.jit
def _embed(idx_t, table):
  mesh = plsc.VectorSubcoreMesh(core_axis_name="c", subcore_axis_name="s")
  k = functools.partial(
      pl.kernel,
      mesh=mesh,
      out_type=jax.ShapeDtypeStruct((SEQ, BATCH, DIM), jnp.float32),
      scratch_types=(
          [pltpu.VMEM((SEQ, COLS_PER_W), jnp.int32)]
          + [pltpu.VMEM((1, COLS_PER_W, DIM), jnp.float32)] * NBUF
          + [pltpu.SemaphoreType.DMA] * (2 * NBUF)
      ),
  )(_emb_body)
  return k(idx_t, table)


def kernel(x, embed_weight):
  idx_t = jnp.swapaxes(x.astype(jnp.int32), 0, 1)
  out_t = _embed(idx_t, embed_weight)
  return jnp.transpose(out_t, (1, 0, 2))


# speedup vs baseline: 3.2312x; 1.0042x over previous
import functools

import jax
import jax.numpy as jnp
from jax import lax
from jax.experimental import pallas as pl
from jax.experimental.pallas import tpu as pltpu
from jax.experimental.pallas import tpu_sc as plsc

VOCAB = 100000
DIM = 128
BATCH = 4096
SEQ = 50
NC = 2
NS = 16
NW = NC * NS
COLS_PER_W = BATCH // NW
NCHUNK = SEQ
NBUF = 5
PF = 2

NSTEADY = (NCHUNK - PF - NBUF) // NBUF
TAIL0 = NBUF + NSTEADY * NBUF

assert PF >= 1 and PF <= NBUF - 2 and NSTEADY >= 1


def _emb_body(idx_hbm, table_hbm, out_hbm, idx_v, *rest):
  bufs = list(rest[:NBUF])
  gsem = list(rest[NBUF:2 * NBUF])
  wsem = list(rest[2 * NBUF:])
  wid = lax.axis_index("s") * NC + lax.axis_index("c")
  col0 = wid * COLS_PER_W
  pltpu.sync_copy(idx_hbm.at[:, pl.ds(col0, COLS_PER_W)], idx_v)

  def start_gather(j, b):
    pltpu.async_copy(table_hbm.at[idx_v.at[j]], bufs[b].at[0], gsem[b])

  def wait_gather(j, b):
    pltpu.make_async_copy(
        table_hbm.at[idx_v.at[j]], bufs[b].at[0], gsem[b]).wait()

  def start_write(j, b):
    pltpu.async_copy(
        bufs[b], out_hbm.at[pl.ds(j, 1), pl.ds(col0, COLS_PER_W)], wsem[b])

  def wait_write(j, b):
    pltpu.make_async_copy(
        bufs[b], out_hbm.at[pl.ds(j, 1), pl.ds(col0, COLS_PER_W)],
        wsem[b]).wait()

  def visit(j, b, pf_wait, pf_gather):
    if pf_gather:
      bf = (b + PF) % NBUF
      if pf_wait:
        wait_write(j + PF - NBUF, bf)
      start_gather(j + PF, bf)
    wait_gather(j, b)
    start_write(j, b)

  for jf in range(PF):
    start_gather(jf, jf % NBUF)
  for b in range(NBUF):
    visit(b, b, pf_wait=(b + PF >= NBUF), pf_gather=True)

  def group(i, _):
    g = i * NBUF
    for b in range(NBUF):
      visit(g + b, b, pf_wait=True, pf_gather=True)
    return 0

  lax.fori_loop(1, 1 + NSTEADY, group, 0)

  for j in range(TAIL0, NCHUNK):
    visit(j, j % NBUF, pf_wait=True, pf_gather=(j + PF < NCHUNK))
  for j in range(NCHUNK - NBUF, NCHUNK):
    wait_write(j, j % NBUF)


@jax.jit
def _embed(idx_t, table):
  mesh = plsc.VectorSubcoreMesh(core_axis_name="c", subcore_axis_name="s")
  k = functools.partial(
      pl.kernel,
      mesh=mesh,
      out_type=jax.ShapeDtypeStruct((SEQ, BATCH, DIM), jnp.float32),
      scratch_types=(
          [pltpu.VMEM((SEQ, COLS_PER_W), jnp.int32)]
          + [pltpu.VMEM((1, COLS_PER_W, DIM), jnp.float32)] * NBUF
          + [pltpu.SemaphoreType.DMA] * (2 * NBUF)
      ),
  )(_emb_body)
  return k(idx_t, table)


def kernel(x, embed_weight):
  idx_t = jnp.swapaxes(x.astype(jnp.int32), 0, 1)
  out_t = _embed(idx_t, embed_weight)
  return jnp.transpose(out_t, (1, 0, 2))


# speedup vs baseline: 3.2606x; 1.0091x over previous
import functools

import jax
import jax.numpy as jnp
from jax import lax
from jax.experimental import pallas as pl
from jax.experimental.pallas import tpu as pltpu
from jax.experimental.pallas import tpu_sc as plsc

VOCAB = 100000
DIM = 128
BATCH = 4096
SEQ = 50
NC = 2
NS = 16
NW = NC * NS
COLS_PER_W = BATCH // NW
NCHUNK = SEQ
NBUF = 7
PF = 4

NSTEADY = (NCHUNK - PF - NBUF) // NBUF
TAIL0 = NBUF + NSTEADY * NBUF

assert PF >= 1 and PF <= NBUF - 2 and NSTEADY >= 1


def _emb_body(idx_hbm, table_hbm, out_hbm, idx_v, *rest):
  bufs = list(rest[:NBUF])
  gsem = list(rest[NBUF:2 * NBUF])
  wsem = list(rest[2 * NBUF:])
  wid = lax.axis_index("s") * NC + lax.axis_index("c")
  col0 = wid * COLS_PER_W
  pltpu.sync_copy(idx_hbm.at[:, pl.ds(col0, COLS_PER_W)], idx_v)

  def start_gather(j, b):
    pltpu.async_copy(table_hbm.at[idx_v.at[j]], bufs[b].at[0], gsem[b])

  def wait_gather(j, b):
    pltpu.make_async_copy(
        table_hbm.at[idx_v.at[j]], bufs[b].at[0], gsem[b]).wait()

  def start_write(j, b):
    pltpu.async_copy(
        bufs[b], out_hbm.at[pl.ds(j, 1), pl.ds(col0, COLS_PER_W)], wsem[b])

  def wait_write(j, b):
    pltpu.make_async_copy(
        bufs[b], out_hbm.at[pl.ds(j, 1), pl.ds(col0, COLS_PER_W)],
        wsem[b]).wait()

  def visit(j, b, pf_wait, pf_gather):
    if pf_gather:
      bf = (b + PF) % NBUF
      if pf_wait:
        wait_write(j + PF - NBUF, bf)
      start_gather(j + PF, bf)
    wait_gather(j, b)
    start_write(j, b)

  for jf in range(PF):
    start_gather(jf, jf % NBUF)
  for b in range(NBUF):
    visit(b, b, pf_wait=(b + PF >= NBUF), pf_gather=True)

  def group(i, _):
    g = i * NBUF
    for b in range(NBUF):
      visit(g + b, b, pf_wait=True, pf_gather=True)
    return 0

  lax.fori_loop(1, 1 + NSTEADY, group, 0)

  for j in range(TAIL0, NCHUNK):
    visit(j, j % NBUF, pf_wait=True, pf_gather=(j + PF < NCHUNK))
  for j in range(NCHUNK - NBUF, NCHUNK):
    wait_write(j, j % NBUF)


@jax.jit
def _embed(idx_t, table):
  mesh = plsc.VectorSubcoreMesh(core_axis_name="c", subcore_axis_name="s")
  k = functools.partial(
      pl.kernel,
      mesh=mesh,
      out_type=jax.ShapeDtypeStruct((SEQ, BATCH, DIM), jnp.float32),
      scratch_types=(
          [pltpu.VMEM((SEQ, COLS_PER_W), jnp.int32)]
          + [pltpu.VMEM((1, COLS_PER_W, DIM), jnp.float32)] * NBUF
          + [pltpu.SemaphoreType.DMA] * (2 * NBUF)
      ),
  )(_emb_body)
  return k(idx_t, table)


def kernel(x, embed_weight):
  idx_t = jnp.swapaxes(x.astype(jnp.int32), 0, 1)
  out_t = _embed(idx_t, embed_weight)
  return jnp.transpose(out_t, (1, 0, 2))
